# baseline (speedup 1.0000x reference)
import functools

import jax
import jax.numpy as jnp
from jax import lax
from jax.experimental import pallas as pl
from jax.experimental.pallas import tpu as pltpu
from jax.experimental.pallas import tpu_sc as plsc

N_V = 10000
N_L = 16
N_EDGES = 160000

NC = 2
NS = 16
NW = NC * NS

ROWS_PER_TILE = 640
PAD_ROWS = NS * ROWS_PER_TILE
SLAB_128 = ROWS_PER_TILE * N_L // 128

TABLE_ROWS = N_V + 16

EDGE_BATCH = 512
BATCHES_PER_W = 10
NBUF = 5
E_PAD = NW * BATCHES_PER_W * EDGE_BATCH
IDX_ROWS = E_PAD // EDGE_BATCH
IDX_ROWS_PER_W = BATCHES_PER_W

_mesh = plsc.VectorSubcoreMesh(core_axis_name="c", subcore_axis_name="s")
_params = pltpu.CompilerParams(use_tc_tiling_on_sc=False)


@functools.partial(
    pl.kernel,
    out_type=jax.ShapeDtypeStruct((NW, N_L), jnp.float32),
    mesh=_mesh,
    scratch_types=(
        [
            pltpu.VMEM((SLAB_128, 128), jnp.float32),
            pltpu.VMEM((ROWS_PER_TILE, N_L), jnp.float32),
            pltpu.VMEM((ROWS_PER_TILE,), jnp.int32),
            pltpu.VMEM((16, N_L), jnp.float32),
            pltpu.VMEM((IDX_ROWS_PER_W, EDGE_BATCH), jnp.int32),
            pltpu.VMEM((IDX_ROWS_PER_W, EDGE_BATCH), jnp.int32),
            pltpu.VMEM((N_L,), jnp.float32),
            pltpu.VMEM_SHARED((TABLE_ROWS, N_L), jnp.float32),
        ]
        + [pltpu.VMEM((EDGE_BATCH, N_L), jnp.float32)] * (2 * NBUF)
        + [pltpu.SemaphoreType.DMA] * (2 * NBUF + 1)
    ),
    compiler_params=_params,
)
def _hoi(slab_hbm, midx_hbm, iab_hbm, out_hbm,
         slab_v, rows_v, idx_v, zero_v, ia_v, ib_v, acc_v, shared, *rest):
    abufs = rest[0:NBUF]
    bbufs = rest[NBUF:2 * NBUF]
    sas = rest[2 * NBUF:3 * NBUF]
    sbs = rest[3 * NBUF:4 * NBUF]
    esa = rest[4 * NBUF]

    sid = lax.axis_index("s")
    wid = sid * NC + lax.axis_index("c")

    row0 = wid * IDX_ROWS_PER_W
    eca = pltpu.async_copy(iab_hbm.at[pl.ds(row0, IDX_ROWS_PER_W)],
                           ia_v, esa)

    pltpu.sync_copy(slab_hbm.at[pl.ds(sid * SLAB_128, SLAB_128)], slab_v)
    pltpu.sync_copy(midx_hbm.at[pl.ds(sid * ROWS_PER_TILE, ROWS_PER_TILE)],
                    idx_v)

    lanes = lax.iota(jnp.int32, N_L)

    def softmax_blk(q, carry):
        for t in range(8):
            e = jnp.exp(slab_v[q, pl.ds(t * N_L, N_L)])
            s = e
            for sh in (8, 4, 2, 1):
                s = s + s.at[lanes ^ sh].get(mode="promise_in_bounds")
            rows_v[q * 8 + t] = e / s
        return carry

    lax.fori_loop(0, SLAB_128, softmax_blk, 0)
    pltpu.sync_copy(rows_v, shared.at[idx_v])

    @pl.when(sid == 0)
    def _():
        def zero_row(i, carry):
            zero_v[i] = jnp.zeros((N_L,), jnp.float32)
            return carry

        lax.fori_loop(0, 16, zero_row, 0)
        pltpu.sync_copy(zero_v, shared.at[pl.ds(N_V, 16)])

    eca.wait()

    def unpack_idx(r, carry):
        for t in range(EDGE_BATCH // N_L):
            v = ia_v[r, pl.ds(t * N_L, N_L)]
            ib_v[r, pl.ds(t * N_L, N_L)] = jnp.right_shift(v, 16)
            ia_v[r, pl.ds(t * N_L, N_L)] = jnp.bitwise_and(v, 0xFFFF)
        return carry

    lax.fori_loop(0, IDX_ROWS_PER_W, unpack_idx, 0)
    plsc.subcore_barrier()

    for b in range(NBUF):
        pltpu.async_copy(shared.at[ia_v.at[b]], abufs[b], sas[b])
        pltpu.async_copy(shared.at[ib_v.at[b]], bbufs[b], sbs[b])

    def outer(g, acc):
        for b in range(NBUF):
            c = g * NBUF + b
            pltpu.make_async_copy(shared.at[ia_v.at[b]], abufs[b],
                                  sas[b]).wait()
            pltpu.make_async_copy(shared.at[ib_v.at[b]], bbufs[b],
                                  sbs[b]).wait()
            a_v, b_v = abufs[b], bbufs[b]

            def edge_fma(j, acc2, a_v=a_v, b_v=b_v):
                return acc2 + a_v[j] * b_v[j]

            acc = lax.fori_loop(0, EDGE_BATCH, edge_fma, acc, unroll=16)

            @pl.when(c + NBUF < BATCHES_PER_W)
            def _(b=b, c=c):
                pltpu.async_copy(shared.at[ia_v.at[c + NBUF]], abufs[b],
                                 sas[b])
                pltpu.async_copy(shared.at[ib_v.at[c + NBUF]], bbufs[b],
                                 sbs[b])
        return acc

    acc = lax.fori_loop(0, BATCHES_PER_W // NBUF, outer,
                        jnp.zeros((N_L,), jnp.float32))
    acc_v[...] = acc
    pltpu.sync_copy(acc_v, out_hbm.at[wid])


def kernel(trainable_params, fixed_params, fixed_indices, trainable_indices,
           simplices_nodes, simplices_edges):
    n_edges = simplices_edges.shape[0]
    merged = jnp.concatenate([trainable_params, fixed_params], axis=0)
    midx = jnp.concatenate([trainable_indices, fixed_indices], axis=0)
    pad = PAD_ROWS - merged.shape[0]
    merged = jnp.concatenate(
        [merged, jnp.broadcast_to(merged[-1:], (pad, N_L))], axis=0)
    midx = jnp.concatenate(
        [midx, jnp.broadcast_to(midx[-1:], (pad,))], axis=0)
    slab = merged.reshape(NS * SLAB_128, 128)

    epad = E_PAD - n_edges
    packed = jnp.bitwise_or(simplices_edges[:, 0],
                            jnp.left_shift(simplices_edges[:, 1], 16))
    fill_val = jnp.int32(N_V | (N_V << 16))
    iab = jnp.concatenate(
        [packed, jnp.full((epad,), fill_val, jnp.int32)]
    ).reshape(IDX_ROWS, EDGE_BATCH)

    partials = _hoi(slab, midx.astype(jnp.int32), iab)
    obj = 2.0 * n_edges - jnp.sum(partials)
    return obj.astype(jnp.float32)


# speedup vs baseline: 1.0754x; 1.0754x over previous
import functools

import jax
import jax.numpy as jnp
from jax import lax
from jax.experimental import pallas as pl
from jax.experimental.pallas import tpu as pltpu
from jax.experimental.pallas import tpu_sc as plsc

N_V = 10000
N_L = 16
N_EDGES = 160000

NC = 2
NS = 16
NW = NC * NS

ROWS_PER_W = 320
PAD_ROWS = NW * ROWS_PER_W
SLAB_128 = ROWS_PER_W * N_L // 128

TABLE_ROWS = N_V + 16
TROWS_PER_TILE = TABLE_ROWS // NS

EDGE_BATCH = 128
BATCHES_PER_W = 40
NBUF = 4
E_PAD = NW * BATCHES_PER_W * EDGE_BATCH
IDX_ROWS = E_PAD // EDGE_BATCH
IDX_ROWS_PER_W = BATCHES_PER_W

_mesh = plsc.VectorSubcoreMesh(core_axis_name="c", subcore_axis_name="s")
_params = pltpu.CompilerParams(use_tc_tiling_on_sc=False)


@functools.partial(
    pl.kernel,
    out_type=jax.ShapeDtypeStruct((TABLE_ROWS, N_L), jnp.float32),
    mesh=_mesh,
    scratch_types=[
        pltpu.VMEM((SLAB_128, 128), jnp.float32),
        pltpu.VMEM((ROWS_PER_W, N_L), jnp.float32),
        pltpu.VMEM((ROWS_PER_W,), jnp.int32),
        pltpu.VMEM((16, N_L), jnp.float32),
        pltpu.SemaphoreType.DMA,
    ],
    compiler_params=_params,
)
def _assemble_softmax(slab_hbm, midx_hbm, table_hbm,
                      slab_v, rows_v, idx_v, zero_v, sem):
    wid = lax.axis_index("s") * NC + lax.axis_index("c")
    pltpu.sync_copy(slab_hbm.at[pl.ds(wid * SLAB_128, SLAB_128)], slab_v)
    pltpu.sync_copy(midx_hbm.at[pl.ds(wid * ROWS_PER_W, ROWS_PER_W)], idx_v)

    lanes = lax.iota(jnp.int32, N_L)

    def softmax_blk(q, carry):
        for t in range(8):
            e = jnp.exp(slab_v[q, pl.ds(t * N_L, N_L)])
            s = e
            for sh in (8, 4, 2, 1):
                s = s + s.at[lanes ^ sh].get(mode="promise_in_bounds")
            rows_v[q * 8 + t] = e / s
        return carry

    lax.fori_loop(0, SLAB_128, softmax_blk, 0)
    pltpu.async_copy(rows_v, table_hbm.at[idx_v], sem).wait()

    @pl.when(wid == 0)
    def _():
        def zero_row(i, carry):
            zero_v[i] = jnp.zeros((N_L,), jnp.float32)
            return carry

        lax.fori_loop(0, 16, zero_row, 0)
        pltpu.sync_copy(zero_v, table_hbm.at[pl.ds(N_V, 16)])


@functools.partial(
    pl.kernel,
    out_type=jax.ShapeDtypeStruct((NW, N_L), jnp.float32),
    mesh=_mesh,
    scratch_types=(
        [
            pltpu.VMEM((IDX_ROWS_PER_W, EDGE_BATCH), jnp.int32),
            pltpu.VMEM((IDX_ROWS_PER_W, EDGE_BATCH), jnp.int32),
            pltpu.VMEM((N_L,), jnp.float32),
            pltpu.VMEM_SHARED((TABLE_ROWS, N_L), jnp.float32),
        ]
        + [pltpu.VMEM((EDGE_BATCH, N_L), jnp.float32)] * (2 * NBUF)
        + [pltpu.SemaphoreType.DMA] * (2 * NBUF + 2)
    ),
    compiler_params=_params,
)
def _edge_dot(table_hbm, iab_hbm, out_hbm, ia_v, ib_v, acc_v, shared, *rest):
    abufs = rest[0:NBUF]
    bbufs = rest[NBUF:2 * NBUF]
    sas = rest[2 * NBUF:3 * NBUF]
    sbs = rest[3 * NBUF:4 * NBUF]
    esa = rest[4 * NBUF]
    tsem = rest[4 * NBUF + 1]

    sid = lax.axis_index("s")
    wid = sid * NC + lax.axis_index("c")

    row0 = wid * IDX_ROWS_PER_W
    eca = pltpu.async_copy(iab_hbm.at[pl.ds(row0, IDX_ROWS_PER_W)],
                           ia_v, esa)
    trow = sid * TROWS_PER_TILE
    tca = pltpu.async_copy(table_hbm.at[pl.ds(trow, TROWS_PER_TILE)],
                           shared.at[pl.ds(trow, TROWS_PER_TILE)], tsem)

    eca.wait()

    def unpack_idx(r, carry):
        for t in range(EDGE_BATCH // N_L):
            v = ia_v[r, pl.ds(t * N_L, N_L)]
            ib_v[r, pl.ds(t * N_L, N_L)] = jnp.right_shift(v, 16)
            ia_v[r, pl.ds(t * N_L, N_L)] = jnp.bitwise_and(v, 0xFFFF)
        return carry

    lax.fori_loop(0, IDX_ROWS_PER_W, unpack_idx, 0)
    tca.wait()
    plsc.subcore_barrier()

    for b in range(NBUF):
        pltpu.async_copy(shared.at[ia_v.at[b]], abufs[b], sas[b])
        pltpu.async_copy(shared.at[ib_v.at[b]], bbufs[b], sbs[b])

    def outer(g, acc):
        for b in range(NBUF):
            c = g * NBUF + b
            pltpu.make_async_copy(shared.at[ia_v.at[b]], abufs[b],
                                  sas[b]).wait()
            pltpu.make_async_copy(shared.at[ib_v.at[b]], bbufs[b],
                                  sbs[b]).wait()
            a_v, b_v = abufs[b], bbufs[b]

            def edge_fma(j, acc2, a_v=a_v, b_v=b_v):
                return acc2 + a_v[j] * b_v[j]

            acc = lax.fori_loop(0, EDGE_BATCH, edge_fma, acc, unroll=16)

            @pl.when(c + NBUF < BATCHES_PER_W)
            def _(b=b, c=c):
                pltpu.async_copy(shared.at[ia_v.at[c + NBUF]], abufs[b],
                                 sas[b])
                pltpu.async_copy(shared.at[ib_v.at[c + NBUF]], bbufs[b],
                                 sbs[b])
        return acc

    acc = lax.fori_loop(0, BATCHES_PER_W // NBUF, outer,
                        jnp.zeros((N_L,), jnp.float32))
    acc_v[...] = acc
    pltpu.sync_copy(acc_v, out_hbm.at[wid])


def kernel(trainable_params, fixed_params, fixed_indices, trainable_indices,
           simplices_nodes, simplices_edges):
    n_edges = simplices_edges.shape[0]
    merged = jnp.concatenate([trainable_params, fixed_params], axis=0)
    midx = jnp.concatenate([trainable_indices, fixed_indices], axis=0)
    pad = PAD_ROWS - merged.shape[0]
    merged = jnp.concatenate(
        [merged, jnp.broadcast_to(merged[-1:], (pad, N_L))], axis=0)
    midx = jnp.concatenate(
        [midx, jnp.broadcast_to(midx[-1:], (pad,))], axis=0)
    slab = merged.reshape(NW * SLAB_128, 128)

    epad = E_PAD - n_edges
    packed = jnp.bitwise_or(simplices_edges[:, 0],
                            jnp.left_shift(simplices_edges[:, 1], 16))
    fill_val = jnp.int32(N_V | (N_V << 16))
    iab = jnp.concatenate(
        [packed, jnp.full((epad,), fill_val, jnp.int32)]
    ).reshape(IDX_ROWS, EDGE_BATCH)

    table = _assemble_softmax(slab, midx.astype(jnp.int32))
    partials = _edge_dot(table, iab)
    obj = 2.0 * n_edges - jnp.sum(partials)
    return obj.astype(jnp.float32)
